# Initial kernel scaffold; baseline (speedup 1.0000x reference)
import functools
import math

import jax
import jax.numpy as jnp
from jax import lax
from jax.experimental import pallas as pl

KERNEL_SIZE_FACTOR = 0.1
SIGMA = 3.0


def _kernel_consts(H):
    ks = int(H * KERNEL_SIZE_FACTOR)
    if ks % 2 == 0:
        ks += 1
    half = ks // 2
    c = (ks - 1) / 2.0
    z = sum(math.exp(-((i - c) ** 2) / (2.0 * SIGMA**2)) for i in range(ks))
    return ks, half, 1.0 / z


def _saliency_tc_kernel(points_ref, out_ref, *, H, W, half, inv_z):
    pts = points_ref[0]
    P = pts.shape[0]
    xs = jnp.floor(pts[:, 0:1] * W)
    ys = jnp.floor(pts[:, 1:2] * H)

    inv_two_sigma2 = -1.0 / (2.0 * SIGMA * SIGMA)

    hio = lax.broadcasted_iota(jnp.float32, (P, H), 1)
    dy = hio - ys
    gy = jnp.where(
        jnp.abs(dy) <= half,
        jnp.exp(dy * dy * inv_two_sigma2) * inv_z,
        0.0,
    )

    wio = lax.broadcasted_iota(jnp.float32, (P, W), 1)
    dx = wio - xs
    gx = jnp.where(
        jnp.abs(dx) <= half,
        jnp.exp(dx * dx * inv_two_sigma2) * inv_z,
        0.0,
    )

    out_ref[0] = lax.dot_general(
        gy, gx, (((0,), (0,)), ((), ())), preferred_element_type=jnp.float32
    )


def kernel(feature_map, points):
    B, C, H, W = feature_map.shape
    P = points.shape[1]
    ks, half, inv_z = _kernel_consts(min(H, W))

    body = functools.partial(
        _saliency_tc_kernel, H=H, W=W, half=half, inv_z=inv_z
    )
    return pl.pallas_call(
        body,
        grid=(B,),
        in_specs=[pl.BlockSpec((1, P, 2), lambda b: (b, 0, 0))],
        out_specs=pl.BlockSpec((1, H, W), lambda b: (b, 0, 0)),
        out_shape=jax.ShapeDtypeStruct((B, H, W), jnp.float32),
    )(points)


# speedup vs baseline: 463.1742x; 463.1742x over previous
import functools
import math

import jax
import jax.numpy as jnp
from jax import lax
from jax.experimental import pallas as pl

KERNEL_SIZE_FACTOR = 0.1
SIGMA = 3.0


def _kernel_consts(H):
    ks = int(H * KERNEL_SIZE_FACTOR)
    if ks % 2 == 0:
        ks += 1
    half = ks // 2
    c = (ks - 1) / 2.0
    z = sum(math.exp(-((i - c) ** 2) / (2.0 * SIGMA**2)) for i in range(ks))
    return ks, half, 1.0 / z


def _saliency_tc_kernel(points_ref, out_ref, *, H, W, half, inv_z):
    pts = points_ref[0]
    P = pts.shape[0]
    xs = jnp.floor(pts[:, 0:1] * W)
    ys = jnp.floor(pts[:, 1:2] * H)

    inv_two_sigma2 = -1.0 / (2.0 * SIGMA * SIGMA)

    hio = lax.broadcasted_iota(jnp.int32, (P, H), 1).astype(jnp.float32)
    dy = hio - ys
    gy = jnp.where(
        jnp.abs(dy) <= half,
        jnp.exp(dy * dy * inv_two_sigma2) * inv_z,
        0.0,
    )

    wio = lax.broadcasted_iota(jnp.int32, (P, W), 1).astype(jnp.float32)
    dx = wio - xs
    gx = jnp.where(
        jnp.abs(dx) <= half,
        jnp.exp(dx * dx * inv_two_sigma2) * inv_z,
        0.0,
    )

    out_ref[0] = lax.dot_general(
        gy, gx, (((0,), (0,)), ((), ())), preferred_element_type=jnp.float32
    )


def kernel(feature_map, points):
    B, C, H, W = feature_map.shape
    P = points.shape[1]
    ks, half, inv_z = _kernel_consts(min(H, W))

    body = functools.partial(
        _saliency_tc_kernel, H=H, W=W, half=half, inv_z=inv_z
    )
    return pl.pallas_call(
        body,
        grid=(B,),
        in_specs=[pl.BlockSpec((1, P, 2), lambda b: (b, 0, 0))],
        out_specs=pl.BlockSpec((1, H, W), lambda b: (b, 0, 0)),
        out_shape=jax.ShapeDtypeStruct((B, H, W), jnp.float32),
    )(points)
